# Initial kernel scaffold; baseline (speedup 1.0000x reference)
import jax
import jax.numpy as jnp
from jax.experimental import pallas as pl


def kernel(feat_a, feat_s, feat_f, feat_p, feat_m, edges_b, edges_c, edges_d, edges_e, edges_j_s, edges_j_f, edges_j_p, edges_i_s, edges_i_f, edges_i_p, params):
    raise NotImplementedError("write your pallas kernel here")


# speedup vs baseline: 1.1118x; 1.1118x over previous
import functools

import jax
import jax.numpy as jnp
from jax import lax
from jax.experimental import pallas as pl
from jax.experimental.pallas import tpu as pltpu
from jax.experimental.pallas import tpu_sc as plsc

HID = 128
HEADS = 8
N = 10000
E = 40000
NPAD = 10240
EPAD = 40960
BM = 512
BM2 = 256

_SAGE = [('b', 'a', 'a'), ('c', 's', 's'), ('d', 'f', 'f'), ('e', 'p', 'p'),
         ('j_s', 'a', 's'), ('j_f', 's', 'f'), ('j_p', 'f', 'p')]
_GAT = [('i_s', 's'), ('i_f', 'f'), ('i_p', 'p')]


def _gather_rows(table, idx):
    V, D = table.shape
    B = idx.shape[0]
    info = plsc.get_sparse_core_info()
    nw = info.num_cores * info.num_subcores
    bpw = B // nw
    ch = 64 if (D >= 512 or bpw % 128 != 0) else 128
    nch = bpw // ch
    mesh = plsc.VectorSubcoreMesh(core_axis_name="c", subcore_axis_name="s")

    @functools.partial(
        pl.kernel, mesh=mesh,
        out_type=jax.ShapeDtypeStruct((B, D), jnp.float32),
        scratch_types=[
            pltpu.VMEM((bpw,), jnp.int32),
            pltpu.VMEM((ch, D), jnp.float32),
            pltpu.SemaphoreType.DMA,
        ])
    def k(table_hbm, idx_hbm, out_hbm, idx_v, rows_v, sem):
        wid = lax.axis_index("s") * info.num_cores + lax.axis_index("c")
        base = wid * bpw
        pltpu.sync_copy(idx_hbm.at[pl.ds(base, bpw)], idx_v)
        for ci in range(nch):
            pltpu.async_copy(
                table_hbm.at[idx_v.at[pl.ds(ci * ch, ch)]], rows_v, sem).wait()
            pltpu.sync_copy(rows_v, out_hbm.at[pl.ds(base + ci * ch, ch)])

    return k(table, idx)


def _prelu(x, av):
    return jnp.where(x >= 0, x, av * x)


def _mm(A, Wt, bias):
    M, K = A.shape
    Ko = Wt.shape[1]

    def kern(a, w, b, o):
        o[...] = jnp.dot(a[...], w[...],
                         preferred_element_type=jnp.float32) + b[...]

    return pl.pallas_call(
        kern, grid=(M // BM,),
        in_specs=[pl.BlockSpec((BM, K), lambda i: (i, 0)),
                  pl.BlockSpec((K, Ko), lambda i: (0, 0)),
                  pl.BlockSpec((1, Ko), lambda i: (0, 0))],
        out_specs=pl.BlockSpec((BM, Ko), lambda i: (i, 0)),
        out_shape=jax.ShapeDtypeStruct((M, Ko), jnp.float32),
    )(A, Wt, bias)


def _mm_prelu(A, Wt, bias, a8):
    M, K = A.shape
    Ko = Wt.shape[1]

    def kern(av_ref, a, w, b, o):
        av = av_ref[0:1, 0:1]
        o[...] = _prelu(jnp.dot(a[...], w[...],
                                preferred_element_type=jnp.float32) + b[...], av)

    return pl.pallas_call(
        kern, grid=(M // BM,),
        in_specs=[pl.BlockSpec((8, 128), lambda i: (0, 0)),
                  pl.BlockSpec((BM, K), lambda i: (i, 0)),
                  pl.BlockSpec((K, Ko), lambda i: (0, 0)),
                  pl.BlockSpec((1, Ko), lambda i: (0, 0))],
        out_specs=pl.BlockSpec((BM, Ko), lambda i: (i, 0)),
        out_shape=jax.ShapeDtypeStruct((M, Ko), jnp.float32),
    )(a8, A, Wt, bias)


def _mm2_acc(acc, A1, W1t, A2, W2t, bias, a8):
    M, K = A1.shape
    Ko = W1t.shape[1]

    def kern(av_ref, acc_ref, a1, w1, a2, w2, b, o):
        av = av_ref[0:1, 0:1]
        s = (jnp.dot(a1[...], w1[...], preferred_element_type=jnp.float32)
             + jnp.dot(a2[...], w2[...], preferred_element_type=jnp.float32)
             + b[...])
        o[...] = acc_ref[...] + _prelu(s, av)

    return pl.pallas_call(
        kern, grid=(M // BM,),
        in_specs=[pl.BlockSpec((8, 128), lambda i: (0, 0)),
                  pl.BlockSpec((BM, Ko), lambda i: (i, 0)),
                  pl.BlockSpec((BM, K), lambda i: (i, 0)),
                  pl.BlockSpec((K, Ko), lambda i: (0, 0)),
                  pl.BlockSpec((BM, K), lambda i: (i, 0)),
                  pl.BlockSpec((K, Ko), lambda i: (0, 0)),
                  pl.BlockSpec((1, Ko), lambda i: (0, 0))],
        out_specs=pl.BlockSpec((BM, Ko), lambda i: (i, 0)),
        out_shape=jax.ShapeDtypeStruct((M, Ko), jnp.float32),
        input_output_aliases={1: 0},
    )(a8, acc, A1, W1t, A2, W2t, bias)


def _lstm_step(t8, cnt8, xg, h, c, res, Whh_t, bhh):
    def kern(t_ref, cnt_ref, xg_ref, h_ref, c_ref, r_ref, w_ref, b_ref,
             ho, co, ro):
        tv = t_ref[0:1, 0:1]
        g = (xg_ref[...]
             + jnp.dot(h_ref[...], w_ref[...],
                       preferred_element_type=jnp.float32)
             + b_ref[...])
        gi = jax.nn.sigmoid(g[:, 0:128])
        gf = jax.nn.sigmoid(g[:, 128:256])
        gg = jnp.tanh(g[:, 256:384])
        go = jax.nn.sigmoid(g[:, 384:512])
        cc = gf * c_ref[...] + gi * gg
        hh = go * jnp.tanh(cc)
        ho[...] = hh
        co[...] = cc
        cap = (cnt_ref[...] == tv + 1.0)[:, 0:1]
        ro[...] = jnp.where(cap, hh, r_ref[...])

    o = jax.ShapeDtypeStruct((NPAD, HID), jnp.float32)
    return pl.pallas_call(
        kern, grid=(NPAD // BM,),
        in_specs=[pl.BlockSpec((8, 128), lambda i: (0, 0)),
                  pl.BlockSpec((BM, 8), lambda i: (i, 0)),
                  pl.BlockSpec((BM, 4 * HID), lambda i: (i, 0)),
                  pl.BlockSpec((BM, HID), lambda i: (i, 0)),
                  pl.BlockSpec((BM, HID), lambda i: (i, 0)),
                  pl.BlockSpec((BM, HID), lambda i: (i, 0)),
                  pl.BlockSpec((HID, 4 * HID), lambda i: (0, 0)),
                  pl.BlockSpec((1, 4 * HID), lambda i: (0, 0))],
        out_specs=[pl.BlockSpec((BM, HID), lambda i: (i, 0))] * 3,
        out_shape=[o, o, o],
        input_output_aliases={3: 0, 4: 1, 5: 2},
    )(t8, cnt8, xg, h, c, res, Whh_t, bhh)


def _gat_max(t8, cnt8, elg, er, m):
    def kern(t_ref, cnt_ref, elg_ref, er_ref, m_ref, o):
        tv = t_ref[0:1, 0:1]
        e = elg_ref[...] + er_ref[...]
        e = jnp.where(e >= 0, e, 0.2 * e)
        mask = tv < cnt_ref[:, 0:1]
        o[...] = jnp.where(mask, jnp.maximum(m_ref[...], e), m_ref[...])

    return pl.pallas_call(
        kern, grid=(NPAD // BM,),
        in_specs=[pl.BlockSpec((8, 128), lambda i: (0, 0)),
                  pl.BlockSpec((BM, 8), lambda i: (i, 0)),
                  pl.BlockSpec((BM, 128), lambda i: (i, 0)),
                  pl.BlockSpec((BM, 128), lambda i: (i, 0)),
                  pl.BlockSpec((BM, 128), lambda i: (i, 0))],
        out_specs=pl.BlockSpec((BM, 128), lambda i: (i, 0)),
        out_shape=jax.ShapeDtypeStruct((NPAD, 128), jnp.float32),
        input_output_aliases={4: 0},
    )(t8, cnt8, elg, er, m)


def _gat_den(t8, cnt8, elg, er, m, den):
    def kern(t_ref, cnt_ref, elg_ref, er_ref, m_ref, d_ref, o):
        tv = t_ref[0:1, 0:1]
        e = elg_ref[...] + er_ref[...]
        e = jnp.where(e >= 0, e, 0.2 * e)
        mask = tv < cnt_ref[:, 0:1]
        o[...] = d_ref[...] + jnp.where(mask, jnp.exp(e - m_ref[...]), 0.0)

    return pl.pallas_call(
        kern, grid=(NPAD // BM,),
        in_specs=[pl.BlockSpec((8, 128), lambda i: (0, 0)),
                  pl.BlockSpec((BM, 8), lambda i: (i, 0))]
        + [pl.BlockSpec((BM, 128), lambda i: (i, 0))] * 4,
        out_specs=pl.BlockSpec((BM, 128), lambda i: (i, 0)),
        out_shape=jax.ShapeDtypeStruct((NPAD, 128), jnp.float32),
        input_output_aliases={5: 0},
    )(t8, cnt8, elg, er, m, den)


def _gat_w(elg_e, nmdg):
    def kern(el_ref, nmd_ref, o):
        e = el_ref[...] + nmd_ref[:, 0:128]
        e = jnp.where(e >= 0, e, 0.2 * e)
        o[...] = jnp.exp(e - nmd_ref[:, 128:256]) / nmd_ref[:, 256:384]

    return pl.pallas_call(
        kern, grid=(EPAD // BM,),
        in_specs=[pl.BlockSpec((BM, 128), lambda i: (i, 0)),
                  pl.BlockSpec((BM, 384), lambda i: (i, 0))],
        out_specs=pl.BlockSpec((BM, 128), lambda i: (i, 0)),
        out_shape=jax.ShapeDtypeStruct((EPAD, 128), jnp.float32),
    )(elg_e, nmdg)


def _gat_accum(t8, cnt8, wg, hsg, seg):
    def kern(t_ref, cnt_ref, wg_ref, hs_ref, s_ref, o):
        tv = t_ref[0:1, 0:1]
        mask = tv < cnt_ref[:, 0:1]
        parts = []
        for h in range(HEADS):
            wcol = jnp.where(mask, wg_ref[:, h:h + 1], 0.0)
            parts.append(s_ref[:, h * HID:(h + 1) * HID]
                         + wcol * hs_ref[:, h * HID:(h + 1) * HID])
        o[...] = jnp.concatenate(parts, axis=1)

    return pl.pallas_call(
        kern, grid=(NPAD // BM2,),
        in_specs=[pl.BlockSpec((8, 128), lambda i: (0, 0)),
                  pl.BlockSpec((BM2, 8), lambda i: (i, 0)),
                  pl.BlockSpec((BM2, 128), lambda i: (i, 0)),
                  pl.BlockSpec((BM2, HEADS * HID), lambda i: (i, 0)),
                  pl.BlockSpec((BM2, HEADS * HID), lambda i: (i, 0))],
        out_specs=pl.BlockSpec((BM2, HEADS * HID), lambda i: (i, 0)),
        out_shape=jax.ShapeDtypeStruct((NPAD, HEADS * HID), jnp.float32),
        input_output_aliases={4: 0},
    )(t8, cnt8, wg, hsg, seg)


def _gat_fin(a8, seg, b1024, macc):
    def kern(av_ref, s_ref, b_ref, m_ref, o):
        av = av_ref[0:1, 0:1]
        o[...] = m_ref[...] + _prelu(s_ref[...] + b_ref[...], av)

    return pl.pallas_call(
        kern, grid=(NPAD // BM2,),
        in_specs=[pl.BlockSpec((8, 128), lambda i: (0, 0)),
                  pl.BlockSpec((BM2, HEADS * HID), lambda i: (i, 0)),
                  pl.BlockSpec((1, HEADS * HID), lambda i: (0, 0)),
                  pl.BlockSpec((BM2, HEADS * HID), lambda i: (i, 0))],
        out_specs=pl.BlockSpec((BM2, HEADS * HID), lambda i: (i, 0)),
        out_shape=jax.ShapeDtypeStruct((NPAD, HEADS * HID), jnp.float32),
        input_output_aliases={3: 0},
    )(a8, seg, b1024, macc)


def _meanlin(a8, x, Wt, bias):
    def kern(av_ref, x_ref, w_ref, b_ref, o):
        av = av_ref[0:1, 0:1]
        xm = x_ref[:, 0:HID]
        for h in range(1, HEADS):
            xm = xm + x_ref[:, h * HID:(h + 1) * HID]
        xm = xm * (1.0 / HEADS)
        o[...] = _prelu(jnp.dot(xm, w_ref[...],
                                preferred_element_type=jnp.float32) + b_ref[...],
                        av)

    return pl.pallas_call(
        kern, grid=(NPAD // BM,),
        in_specs=[pl.BlockSpec((8, 128), lambda i: (0, 0)),
                  pl.BlockSpec((BM, HEADS * HID), lambda i: (i, 0)),
                  pl.BlockSpec((HID, HID), lambda i: (0, 0)),
                  pl.BlockSpec((1, HID), lambda i: (0, 0))],
        out_specs=pl.BlockSpec((BM, HID), lambda i: (i, 0)),
        out_shape=jax.ShapeDtypeStruct((NPAD, HID), jnp.float32),
    )(a8, x, Wt, bias)


def _sort_edges(src, dst):
    order = jnp.argsort(dst, stable=True)
    ss = src[order].astype(jnp.int32)
    dsts = dst[order].astype(jnp.int32)
    counts = jnp.bincount(dst, length=N).astype(jnp.int32)
    off = (jnp.cumsum(counts) - counts).astype(jnp.int32)
    ss_pad = jnp.pad(ss, (0, EPAD - E))
    dst_pad = jnp.pad(dsts, (0, EPAD - E))
    counts_p = jnp.pad(counts, (0, NPAD - N))
    off_p = jnp.pad(off, (0, NPAD - N))
    return ss_pad, dst_pad, counts_p, off_p, jnp.max(counts)


def _run_sage(fsrc_p, src, dst, p):
    ss_pad, _, counts_p, off_p, maxdeg = _sort_edges(src, dst)
    cnt8 = jnp.broadcast_to(counts_p.astype(jnp.float32)[:, None], (NPAD, 8))
    X = _mm(fsrc_p, p['W_ih'].T, p['b_ih'][None])
    Whh_t = p['W_hh'].T
    bhh = p['b_hh'][None]
    z = jnp.zeros((NPAD, HID), jnp.float32)

    def body(cr):
        t, h, c, r = cr
        idx = ss_pad[jnp.minimum(off_p + t, EPAD - 1)]
        xg = _gather_rows(X, idx)
        t8 = jnp.full((8, 128), t, jnp.float32)
        h, c, r = _lstm_step(t8, cnt8, xg, h, c, r, Whh_t, bhh)
        return (t + 1, h, c, r)

    _, _, _, r = lax.while_loop(lambda cr: cr[0] < maxdeg, body,
                                (jnp.int32(0), z, z, z))
    return r


def _run_gat(macc, fsrc_p, fm_p, src, dst, p, a8):
    ss_pad, dst_pad, counts_p, off_p, maxdeg = _sort_edges(src, dst)
    cnt8 = jnp.broadcast_to(counts_p.astype(jnp.float32)[:, None], (NPAD, 8))
    hs = _mm(fsrc_p, p['W'], jnp.zeros((1, HEADS * HID), jnp.float32))
    W3 = p['W'].reshape(HID, HEADS, HID)
    Wl = jnp.pad(jnp.einsum('khd,hd->kh', W3, p['attn_l']), ((0, 0), (0, 120)))
    Wr = jnp.pad(jnp.einsum('khd,hd->kh', W3, p['attn_r']), ((0, 0), (0, 120)))
    z128 = jnp.zeros((1, 128), jnp.float32)
    el = _mm(fsrc_p, Wl, z128)
    er = _mm(fm_p, Wr, z128)

    def body_m(cr):
        t, m = cr
        idx = ss_pad[jnp.minimum(off_p + t, EPAD - 1)]
        elg = _gather_rows(el, idx)
        t8 = jnp.full((8, 128), t, jnp.float32)
        return (t + 1, _gat_max(t8, cnt8, elg, er, m))

    _, m = lax.while_loop(lambda cr: cr[0] < maxdeg, body_m,
                          (jnp.int32(0), jnp.full((NPAD, 128), -jnp.inf,
                                                  jnp.float32)))

    def body_d(cr):
        t, den = cr
        idx = ss_pad[jnp.minimum(off_p + t, EPAD - 1)]
        elg = _gather_rows(el, idx)
        t8 = jnp.full((8, 128), t, jnp.float32)
        return (t + 1, _gat_den(t8, cnt8, elg, er, m, den))

    _, den = lax.while_loop(lambda cr: cr[0] < maxdeg, body_d,
                            (jnp.int32(0), jnp.zeros((NPAD, 128), jnp.float32)))

    nmd = jnp.concatenate([er, m, den], axis=1)
    elg_e = _gather_rows(el, ss_pad)
    nmdg = _gather_rows(nmd, dst_pad)
    w = _gat_w(elg_e, nmdg)

    def body_a(cr):
        t, seg = cr
        idxw = jnp.minimum(off_p + t, EPAD - 1)
        idxh = ss_pad[idxw]
        wg = _gather_rows(w, idxw)
        hsg = _gather_rows(hs, idxh)
        t8 = jnp.full((8, 128), t, jnp.float32)
        return (t + 1, _gat_accum(t8, cnt8, wg, hsg, seg))

    _, seg = lax.while_loop(lambda cr: cr[0] < maxdeg, body_a,
                            (jnp.int32(0),
                             jnp.zeros((NPAD, HEADS * HID), jnp.float32)))
    return _gat_fin(a8, seg, p['b'][None], macc)


def kernel(feat_a, feat_s, feat_f, feat_p, feat_m, edges_b, edges_c, edges_d,
           edges_e, edges_j_s, edges_j_f, edges_j_p, edges_i_s, edges_i_f,
           edges_i_p, params):
    feats = {'a': feat_a, 's': feat_s, 'f': feat_f, 'p': feat_p, 'm': feat_m}
    fpads = {k: jnp.pad(v, ((0, NPAD - N), (0, 0))) for k, v in feats.items()}
    edges = {'b': edges_b, 'c': edges_c, 'd': edges_d, 'e': edges_e,
             'j_s': edges_j_s, 'j_f': edges_j_f, 'j_p': edges_j_p,
             'i_s': edges_i_s, 'i_f': edges_i_f, 'i_p': edges_i_p}
    alpha = params['prelu']
    a8 = jnp.full((8, 128), alpha, jnp.float32)

    acc = {}
    for et, st, dt in _SAGE:
        p = params['sage'][et]
        r = _run_sage(fpads[st], edges[et][0], edges[et][1], p)
        prev = acc.get(dt, jnp.zeros((NPAD, HID), jnp.float32))
        acc[dt] = _mm2_acc(prev, fpads[dt], p['W_self'].T, r, p['W_neigh'].T,
                           p['b'][None], a8)

    macc = jnp.zeros((NPAD, HEADS * HID), jnp.float32)
    for et, st in _GAT:
        macc = _run_gat(macc, fpads[st], fpads['m'], edges[et][0],
                        edges[et][1], params['gat'][et], a8)
    out_m = _meanlin(a8, macc, params['lin']['m']['W'].T,
                     params['lin']['m']['b'][None])

    outs = []
    for nt in ['a', 's', 'f', 'p']:
        lp = params['lin'][nt]
        outs.append(_mm_prelu(acc[nt], lp['W'].T, lp['b'][None], a8)[:N])
    return (outs[0], outs[1], outs[2], outs[3], out_m[:N])
